# Initial kernel scaffold; baseline (speedup 1.0000x reference)
import jax
import jax.numpy as jnp
from jax.experimental import pallas as pl


def kernel(x, edge_index, W1, b1, g1, be1, W2, b2, g2, be2, W3, b3, g3, be3, Wl1, bl1, g4, be4, Wl2, bl2):
    raise NotImplementedError("write your pallas kernel here")


# speedup vs baseline: 15.9691x; 15.9691x over previous
import functools

import jax
import jax.numpy as jnp
from jax import lax
from jax.experimental import pallas as pl
from jax.experimental.pallas import tpu as pltpu
from jax.experimental.pallas import tpu_sc as plsc

N = 10000
NPAD = 10240
E = 320000
EPAD = 327680
NROWS = EPAD // 128
ROWS_PER_TILE = NPAD // 16
BN = 1280
F32 = jnp.float32

_MESH = plsc.VectorSubcoreMesh(core_axis_name="c", subcore_axis_name="s")


def _deg_kernel(dst2d):
    nj = NROWS // 16

    @functools.partial(
        pl.kernel, mesh=_MESH,
        out_type=jax.ShapeDtypeStruct((NPAD,), F32),
        scratch_types=[
            pltpu.VMEM((nj, 128), jnp.int32),
            pltpu.VMEM((128,), F32),
            pltpu.VMEM((ROWS_PER_TILE,), F32),
            pltpu.VMEM_SHARED((NPAD,), F32),
            pltpu.SemaphoreType.DMA,
        ],
    )
    def k(dst_hbm, out_hbm, dst_v, ones_v, ztile_v, deg_sh, sem):
        cid = lax.axis_index("c")
        sid = lax.axis_index("s")
        base = sid * ROWS_PER_TILE

        @pl.when(cid == 0)
        def _zero():
            @pl.loop(0, ROWS_PER_TILE // 16)
            def _(i):
                ztile_v[pl.ds(i * 16, 16)] = jnp.zeros((16,), F32)

            pltpu.sync_copy(ztile_v, deg_sh.at[pl.ds(base, ROWS_PER_TILE)])

        plsc.subcore_barrier()

        @pl.when(cid == 0)
        def _scatter():
            pltpu.sync_copy(dst_hbm.at[pl.ds(sid * nj, nj)], dst_v)

            @pl.loop(0, 8)
            def _(i):
                ones_v[pl.ds(i * 16, 16)] = jnp.ones((16,), F32)

            @pl.loop(0, nj)
            def _(j):
                pltpu.sync_copy(ones_v, deg_sh.at[dst_v.at[j]], add=True)

        plsc.subcore_barrier()

        @pl.when(cid == 0)
        def _out():
            pltpu.sync_copy(deg_sh.at[pl.ds(base, ROWS_PER_TILE)], ztile_v)
            pltpu.sync_copy(ztile_v, out_hbm.at[pl.ds(base, ROWS_PER_TILE)])

    return k(dst2d)


def _conv_kernel(tables, src2d, dst2d, zeros2d, *, split_edges):
    nj = NROWS // 32 if split_edges else NROWS // 16
    G = 16

    @functools.partial(
        pl.kernel, mesh=_MESH,
        out_type=jax.ShapeDtypeStruct((2, NPAD, 128), F32),
        scratch_types=[
            pltpu.VMEM((G, 128), jnp.int32),
            pltpu.VMEM((G, 128), jnp.int32),
            pltpu.VMEM((128, 128), F32),
            pltpu.VMEM_SHARED((NPAD, 128), F32),
            pltpu.SemaphoreType.DMA,
        ],
    )
    def k(tab_hbm, src_hbm, dst_hbm, z_hbm, out_hbm,
          src_v, dst_v, rows_v, acc_sh, sem):
        cid = lax.axis_index("c")
        sid = lax.axis_index("s")
        base = sid * ROWS_PER_TILE

        pltpu.sync_copy(z_hbm, rows_v)

        @pl.loop(0, ROWS_PER_TILE // 128)
        def _(t):
            pltpu.sync_copy(rows_v, acc_sh.at[pl.ds(base + t * 128, 128)])

        plsc.subcore_barrier()

        if split_edges:
            erow = (cid * 16 + sid) * nj
            tab = tab_hbm.at[0]
        else:
            erow = sid * nj
            tab = tab_hbm.at[cid]

        @pl.loop(0, nj // G)
        def _(g):
            row0 = erow + g * G
            pltpu.sync_copy(src_hbm.at[pl.ds(row0, G)], src_v)
            pltpu.sync_copy(dst_hbm.at[pl.ds(row0, G)], dst_v)

            @pl.loop(0, G)
            def _(j):
                pltpu.async_copy(tab.at[src_v.at[j]], rows_v, sem).wait()
                pltpu.sync_copy(rows_v, acc_sh.at[dst_v.at[j]], add=True)

        plsc.subcore_barrier()

        @pl.loop(0, ROWS_PER_TILE // 128)
        def _(t):
            r0 = base + t * 128
            pltpu.sync_copy(acc_sh.at[pl.ds(r0, 128)],
                            out_hbm.at[cid].at[pl.ds(r0, 128)])

    return k(tables, src2d, dst2d, zeros2d)


def _ln(x, g, b, eps=1e-5):
    mu = jnp.mean(x, axis=-1, keepdims=True)
    var = jnp.mean(jnp.square(x - mu), axis=-1, keepdims=True)
    return (x - mu) * lax.rsqrt(var + eps) * g + b


def _elu(x):
    return jnp.where(x > 0, x, jnp.exp(x) - 1.0)


def _row_spec(f):
    return pl.BlockSpec((BN, f), lambda i: (i, 0))


def _full_spec(shape):
    return pl.BlockSpec(shape, lambda i: tuple(0 for _ in shape))


def _stack_spec():
    return pl.BlockSpec((2, BN, 128), lambda i: (0, i, 0))


def _tc1(deg_col, x, W1):
    def body(deg_r, x_r, w_r, dinv_r, p_r):
        dinv = lax.rsqrt(deg_r[...] + 1.0)
        dinv_r[...] = dinv
        p_r[...] = dinv * jnp.dot(x_r[...], w_r[...],
                                  preferred_element_type=F32)

    return pl.pallas_call(
        body,
        grid=(NPAD // BN,),
        in_specs=[_row_spec(1), _row_spec(128), _full_spec((128, 128))],
        out_specs=[_row_spec(1), _row_spec(128)],
        out_shape=[jax.ShapeDtypeStruct((NPAD, 1), F32),
                   jax.ShapeDtypeStruct((NPAD, 128), F32)],
    )(deg_col, x, W1)


def _tc2(dinv_col, s1, p1, b1, g1, be1, W2):
    def body(dinv_r, s_r, p_r, b_r, g_r, be_r, w_r, o_r):
        dinv = dinv_r[...]
        pre = dinv * (s_r[0] + s_r[1] + p_r[...]) + b_r[...]
        h = _elu(_ln(pre, g_r[...], be_r[...]))
        p2 = dinv * jnp.dot(h, w_r[...], preferred_element_type=F32)
        o_r[0] = p2[:, :128]
        o_r[1] = p2[:, 128:]

    return pl.pallas_call(
        body,
        grid=(NPAD // BN,),
        in_specs=[_row_spec(1), _stack_spec(), _row_spec(128),
                  _full_spec((1, 128)), _full_spec((1, 128)),
                  _full_spec((1, 128)), _full_spec((128, 256))],
        out_specs=_stack_spec(),
        out_shape=jax.ShapeDtypeStruct((2, NPAD, 128), F32),
    )(dinv_col, s1, p1, b1, g1, be1, W2)


def _tc3(dinv_col, s2, p2, b2, g2, be2, W3):
    def body(dinv_r, s_r, p_r, b_r, g_r, be_r, w_r, o_r):
        dinv = dinv_r[...]
        pre = jnp.concatenate([s_r[0] + p_r[0], s_r[1] + p_r[1]], axis=-1)
        pre = dinv * pre + b_r[...]
        h = _elu(_ln(pre, g_r[...], be_r[...]))
        o_r[...] = dinv * jnp.dot(h, w_r[...], preferred_element_type=F32)

    return pl.pallas_call(
        body,
        grid=(NPAD // BN,),
        in_specs=[_row_spec(1), _stack_spec(), _stack_spec(),
                  _full_spec((1, 256)), _full_spec((1, 256)),
                  _full_spec((1, 256)), _full_spec((256, 128))],
        out_specs=_row_spec(128),
        out_shape=jax.ShapeDtypeStruct((NPAD, 128), F32),
    )(dinv_col, s2, p2, b2, g2, be2, W3)


def _tc4(dinv_col, s3, p3, b3, g3, be3, Wl1, bl1, g4, be4, Wl2, bl2):
    def body(dinv_r, s_r, p_r, b_r, g_r, be_r, w1_r, b1_r, g4_r, be4_r,
             w2_r, b2_r, o_r):
        dinv = dinv_r[...]
        pre = dinv * (s_r[0] + s_r[1] + p_r[...]) + b_r[...]
        h = _elu(_ln(pre, g_r[...], be_r[...]))
        h = jnp.dot(h, w1_r[...], preferred_element_type=F32) + b1_r[...]
        h = _elu(_ln(h, g4_r[...], be4_r[...]))
        o_r[...] = jnp.dot(h, w2_r[...], preferred_element_type=F32) + b2_r[...]

    return pl.pallas_call(
        body,
        grid=(NPAD // BN,),
        in_specs=[_row_spec(1), _stack_spec(), _row_spec(128),
                  _full_spec((1, 128)), _full_spec((1, 128)),
                  _full_spec((1, 128)), _full_spec((128, 64)),
                  _full_spec((1, 64)), _full_spec((1, 64)),
                  _full_spec((1, 64)), _full_spec((64, 64)),
                  _full_spec((1, 64))],
        out_specs=_row_spec(64),
        out_shape=jax.ShapeDtypeStruct((NPAD, 64), F32),
    )(dinv_col, s3, p3, b3, g3, be3, Wl1, bl1, g4, be4, Wl2, bl2)


def kernel(x, edge_index, W1, b1, g1, be1, W2, b2, g2, be2, W3, b3, g3, be3,
           Wl1, bl1, g4, be4, Wl2, bl2):
    pad = EPAD - E
    pad_idx = N + (jnp.arange(pad, dtype=jnp.int32) % 128)
    src2d = jnp.concatenate([edge_index[0], pad_idx]).reshape(NROWS, 128)
    dst2d = jnp.concatenate([edge_index[1], pad_idx]).reshape(NROWS, 128)
    zeros2d = jnp.zeros((128, 128), F32)
    x_pad = jnp.pad(x, ((0, NPAD - N), (0, 0)))

    row = lambda v: v.reshape(1, -1)

    deg = _deg_kernel(dst2d)
    dinv_col, p1 = _tc1(deg.reshape(NPAD, 1), x_pad, W1)
    s1 = _conv_kernel(p1[None], src2d, dst2d, zeros2d, split_edges=True)
    p2 = _tc2(dinv_col, s1, p1, row(b1), row(g1), row(be1), W2)
    s2 = _conv_kernel(p2, src2d, dst2d, zeros2d, split_edges=False)
    p3 = _tc3(dinv_col, s2, p2, row(b2), row(g2), row(be2), W3)
    s3 = _conv_kernel(p3[None], src2d, dst2d, zeros2d, split_edges=True)
    out = _tc4(dinv_col, s3, p3, row(b3), row(g3), row(be3),
               Wl1, row(bl1), row(g4), row(be4), Wl2, row(bl2))
    return out[:N]


# speedup vs baseline: 21.2468x; 1.3305x over previous
import functools

import jax
import jax.numpy as jnp
from jax import lax
from jax.experimental import pallas as pl
from jax.experimental.pallas import tpu as pltpu
from jax.experimental.pallas import tpu_sc as plsc

N = 10000
NPAD = 10240
E = 320000
EPAD = 327680
NROWS = EPAD // 128
ROWS_PER_TILE = NPAD // 16
BN = 1280
F32 = jnp.float32

_MESH = plsc.VectorSubcoreMesh(core_axis_name="c", subcore_axis_name="s")


def _deg_kernel(dst2d):
    nj = NROWS // 16

    @functools.partial(
        pl.kernel, mesh=_MESH,
        out_type=jax.ShapeDtypeStruct((NPAD,), F32),
        scratch_types=[
            pltpu.VMEM((nj, 128), jnp.int32),
            pltpu.VMEM((128,), F32),
            pltpu.VMEM((ROWS_PER_TILE,), F32),
            pltpu.VMEM_SHARED((NPAD,), F32),
            pltpu.SemaphoreType.DMA,
        ],
    )
    def k(dst_hbm, out_hbm, dst_v, ones_v, ztile_v, deg_sh, sem):
        cid = lax.axis_index("c")
        sid = lax.axis_index("s")
        base = sid * ROWS_PER_TILE

        @pl.when(cid == 0)
        def _zero():
            @pl.loop(0, ROWS_PER_TILE // 16)
            def _(i):
                ztile_v[pl.ds(i * 16, 16)] = jnp.zeros((16,), F32)

            pltpu.sync_copy(ztile_v, deg_sh.at[pl.ds(base, ROWS_PER_TILE)])

        plsc.subcore_barrier()

        @pl.when(cid == 0)
        def _scatter():
            pltpu.sync_copy(dst_hbm.at[pl.ds(sid * nj, nj)], dst_v)

            @pl.loop(0, 8)
            def _(i):
                ones_v[pl.ds(i * 16, 16)] = jnp.ones((16,), F32)

            @pl.loop(0, nj)
            def _(j):
                pltpu.sync_copy(ones_v, deg_sh.at[dst_v.at[j]], add=True)

        plsc.subcore_barrier()

        @pl.when(cid == 0)
        def _out():
            pltpu.sync_copy(deg_sh.at[pl.ds(base, ROWS_PER_TILE)], ztile_v)
            pltpu.sync_copy(ztile_v, out_hbm.at[pl.ds(base, ROWS_PER_TILE)])

    return k(dst2d)


def _conv_kernel(tables, src2d, dst2d, zeros2d, *, split_edges):
    nj = NROWS // 32 if split_edges else NROWS // 16
    G = 8

    @functools.partial(
        pl.kernel, mesh=_MESH,
        out_type=jax.ShapeDtypeStruct((2, NPAD, 128), F32),
        scratch_types=[
            pltpu.VMEM((G, 128), jnp.int32),
            pltpu.VMEM((G, 128), jnp.int32),
            pltpu.VMEM((128, 128), F32),
            pltpu.VMEM((128, 128), F32),
            pltpu.VMEM_SHARED((NPAD, 128), F32),
            pltpu.SemaphoreType.DMA,
            pltpu.SemaphoreType.DMA,
            pltpu.SemaphoreType.DMA,
            pltpu.SemaphoreType.DMA,
        ],
    )
    def k(tab_hbm, src_hbm, dst_hbm, z_hbm, out_hbm,
          src_v, dst_v, rows0, rows1, acc_sh, gs0, gs1, ss0, ss1):
        cid = lax.axis_index("c")
        sid = lax.axis_index("s")
        base = sid * ROWS_PER_TILE

        pltpu.sync_copy(z_hbm, rows0)

        @pl.loop(0, ROWS_PER_TILE // 128)
        def _(t):
            pltpu.sync_copy(rows0, acc_sh.at[pl.ds(base + t * 128, 128)])

        plsc.subcore_barrier()

        if split_edges:
            erow = (cid * 16 + sid) * nj
            tab = tab_hbm.at[0]
        else:
            erow = sid * nj
            tab = tab_hbm.at[cid]

        bufs = (rows0, rows1)
        gsem = (gs0, gs1)
        ssem = (ss0, ss1)

        @pl.loop(0, nj // G)
        def _(g):
            row0 = erow + g * G
            pltpu.sync_copy(src_hbm.at[pl.ds(row0, G)], src_v)
            pltpu.sync_copy(dst_hbm.at[pl.ds(row0, G)], dst_v)
            gd = [None, None]
            sd = [None, None]
            gd[0] = pltpu.async_copy(tab.at[src_v.at[0]], bufs[0], gsem[0])
            for j in range(G):
                b = j & 1
                if j + 1 < G:
                    nb = (j + 1) & 1
                    if sd[nb] is not None:
                        sd[nb].wait()
                    gd[nb] = pltpu.async_copy(tab.at[src_v.at[j + 1]],
                                              bufs[nb], gsem[nb])
                gd[b].wait()
                sd[b] = pltpu.async_copy(bufs[b], acc_sh.at[dst_v.at[j]],
                                         ssem[b], add=True)
            sd[0].wait()
            sd[1].wait()

        plsc.subcore_barrier()

        @pl.loop(0, ROWS_PER_TILE // 128)
        def _(t):
            r0 = base + t * 128
            pltpu.sync_copy(acc_sh.at[pl.ds(r0, 128)],
                            out_hbm.at[cid].at[pl.ds(r0, 128)])

    return k(tables, src2d, dst2d, zeros2d)


def _ln(x, g, b, eps=1e-5):
    mu = jnp.mean(x, axis=-1, keepdims=True)
    var = jnp.mean(jnp.square(x - mu), axis=-1, keepdims=True)
    return (x - mu) * lax.rsqrt(var + eps) * g + b


def _elu(x):
    return jnp.where(x > 0, x, jnp.exp(x) - 1.0)


def _row_spec(f):
    return pl.BlockSpec((BN, f), lambda i: (i, 0))


def _full_spec(shape):
    return pl.BlockSpec(shape, lambda i: tuple(0 for _ in shape))


def _stack_spec():
    return pl.BlockSpec((2, BN, 128), lambda i: (0, i, 0))


def _tc1(deg_col, x, W1):
    def body(deg_r, x_r, w_r, dinv_r, p_r):
        dinv = lax.rsqrt(deg_r[...] + 1.0)
        dinv_r[...] = dinv
        p_r[...] = dinv * jnp.dot(x_r[...], w_r[...],
                                  preferred_element_type=F32)

    return pl.pallas_call(
        body,
        grid=(NPAD // BN,),
        in_specs=[_row_spec(1), _row_spec(128), _full_spec((128, 128))],
        out_specs=[_row_spec(1), _row_spec(128)],
        out_shape=[jax.ShapeDtypeStruct((NPAD, 1), F32),
                   jax.ShapeDtypeStruct((NPAD, 128), F32)],
    )(deg_col, x, W1)


def _tc2(dinv_col, s1, p1, b1, g1, be1, W2):
    def body(dinv_r, s_r, p_r, b_r, g_r, be_r, w_r, o_r):
        dinv = dinv_r[...]
        pre = dinv * (s_r[0] + s_r[1] + p_r[...]) + b_r[...]
        h = _elu(_ln(pre, g_r[...], be_r[...]))
        p2 = dinv * jnp.dot(h, w_r[...], preferred_element_type=F32)
        o_r[0] = p2[:, :128]
        o_r[1] = p2[:, 128:]

    return pl.pallas_call(
        body,
        grid=(NPAD // BN,),
        in_specs=[_row_spec(1), _stack_spec(), _row_spec(128),
                  _full_spec((1, 128)), _full_spec((1, 128)),
                  _full_spec((1, 128)), _full_spec((128, 256))],
        out_specs=_stack_spec(),
        out_shape=jax.ShapeDtypeStruct((2, NPAD, 128), F32),
    )(dinv_col, s1, p1, b1, g1, be1, W2)


def _tc3(dinv_col, s2, p2, b2, g2, be2, W3):
    def body(dinv_r, s_r, p_r, b_r, g_r, be_r, w_r, o_r):
        dinv = dinv_r[...]
        pre = jnp.concatenate([s_r[0] + p_r[0], s_r[1] + p_r[1]], axis=-1)
        pre = dinv * pre + b_r[...]
        h = _elu(_ln(pre, g_r[...], be_r[...]))
        o_r[...] = dinv * jnp.dot(h, w_r[...], preferred_element_type=F32)

    return pl.pallas_call(
        body,
        grid=(NPAD // BN,),
        in_specs=[_row_spec(1), _stack_spec(), _stack_spec(),
                  _full_spec((1, 256)), _full_spec((1, 256)),
                  _full_spec((1, 256)), _full_spec((256, 128))],
        out_specs=_row_spec(128),
        out_shape=jax.ShapeDtypeStruct((NPAD, 128), F32),
    )(dinv_col, s2, p2, b2, g2, be2, W3)


def _tc4(dinv_col, s3, p3, b3, g3, be3, Wl1, bl1, g4, be4, Wl2, bl2):
    def body(dinv_r, s_r, p_r, b_r, g_r, be_r, w1_r, b1_r, g4_r, be4_r,
             w2_r, b2_r, o_r):
        dinv = dinv_r[...]
        pre = dinv * (s_r[0] + s_r[1] + p_r[...]) + b_r[...]
        h = _elu(_ln(pre, g_r[...], be_r[...]))
        h = jnp.dot(h, w1_r[...], preferred_element_type=F32) + b1_r[...]
        h = _elu(_ln(h, g4_r[...], be4_r[...]))
        o_r[...] = jnp.dot(h, w2_r[...], preferred_element_type=F32) + b2_r[...]

    return pl.pallas_call(
        body,
        grid=(NPAD // BN,),
        in_specs=[_row_spec(1), _stack_spec(), _row_spec(128),
                  _full_spec((1, 128)), _full_spec((1, 128)),
                  _full_spec((1, 128)), _full_spec((128, 64)),
                  _full_spec((1, 64)), _full_spec((1, 64)),
                  _full_spec((1, 64)), _full_spec((64, 64)),
                  _full_spec((1, 64))],
        out_specs=_row_spec(64),
        out_shape=jax.ShapeDtypeStruct((NPAD, 64), F32),
    )(dinv_col, s3, p3, b3, g3, be3, Wl1, bl1, g4, be4, Wl2, bl2)


def kernel(x, edge_index, W1, b1, g1, be1, W2, b2, g2, be2, W3, b3, g3, be3,
           Wl1, bl1, g4, be4, Wl2, bl2):
    pad = EPAD - E
    pad_idx = N + (jnp.arange(pad, dtype=jnp.int32) % 128)
    src2d = jnp.concatenate([edge_index[0], pad_idx]).reshape(NROWS, 128)
    dst2d = jnp.concatenate([edge_index[1], pad_idx]).reshape(NROWS, 128)
    zeros2d = jnp.zeros((128, 128), F32)
    x_pad = jnp.pad(x, ((0, NPAD - N), (0, 0)))

    row = lambda v: v.reshape(1, -1)

    deg = _deg_kernel(dst2d)
    dinv_col, p1 = _tc1(deg.reshape(NPAD, 1), x_pad, W1)
    s1 = _conv_kernel(p1[None], src2d, dst2d, zeros2d, split_edges=True)
    p2 = _tc2(dinv_col, s1, p1, row(b1), row(g1), row(be1), W2)
    s2 = _conv_kernel(p2, src2d, dst2d, zeros2d, split_edges=False)
    p3 = _tc3(dinv_col, s2, p2, row(b2), row(g2), row(be2), W3)
    s3 = _conv_kernel(p3[None], src2d, dst2d, zeros2d, split_edges=True)
    out = _tc4(dinv_col, s3, p3, row(b3), row(g3), row(be3),
               Wl1, row(bl1), row(g4), row(be4), Wl2, row(bl2))
    return out[:N]


# speedup vs baseline: 24.4148x; 1.1491x over previous
import functools

import jax
import jax.numpy as jnp
from jax import lax
from jax.experimental import pallas as pl
from jax.experimental.pallas import tpu as pltpu
from jax.experimental.pallas import tpu_sc as plsc

N = 10000
NPAD = 10240
E = 320000
EPAD = 327680
NROWS = EPAD // 128
NJ = NROWS // 32
ROWS_PER_TILE = NPAD // 16
BN = 1280
F32 = jnp.float32

_MESH = plsc.VectorSubcoreMesh(core_axis_name="c", subcore_axis_name="s")


def _deg_kernel(dst2d):
    @functools.partial(
        pl.kernel, mesh=_MESH,
        out_type=[jax.ShapeDtypeStruct((NPAD,), F32),
                  jax.ShapeDtypeStruct((NPAD,), F32)],
        scratch_types=[
            pltpu.VMEM((NJ, 128), jnp.int32),
            pltpu.VMEM((128,), F32),
            pltpu.VMEM((ROWS_PER_TILE,), F32),
            pltpu.VMEM_SHARED((NPAD,), F32),
            pltpu.SemaphoreType.DMA,
        ],
    )
    def k(dst_hbm, out0_hbm, out1_hbm, dst_v, ones_v, ztile_v, deg_sh, sem):
        cid = lax.axis_index("c")
        sid = lax.axis_index("s")
        base = sid * ROWS_PER_TILE
        erow = (cid * 16 + sid) * NJ

        @pl.loop(0, ROWS_PER_TILE // 16)
        def _(i):
            ztile_v[pl.ds(i * 16, 16)] = jnp.zeros((16,), F32)

        pltpu.sync_copy(ztile_v, deg_sh.at[pl.ds(base, ROWS_PER_TILE)])
        pltpu.sync_copy(dst_hbm.at[pl.ds(erow, NJ)], dst_v)

        @pl.loop(0, 8)
        def _(i):
            ones_v[pl.ds(i * 16, 16)] = jnp.ones((16,), F32)

        plsc.subcore_barrier()

        @pl.loop(0, NJ)
        def _(j):
            pltpu.sync_copy(ones_v, deg_sh.at[dst_v.at[j]], add=True)

        plsc.subcore_barrier()

        @pl.when(cid == 0)
        def _():
            pltpu.sync_copy(deg_sh.at[pl.ds(base, ROWS_PER_TILE)], ztile_v)
            pltpu.sync_copy(ztile_v, out0_hbm.at[pl.ds(base, ROWS_PER_TILE)])

        @pl.when(cid == 1)
        def _():
            pltpu.sync_copy(deg_sh.at[pl.ds(base, ROWS_PER_TILE)], ztile_v)
            pltpu.sync_copy(ztile_v, out1_hbm.at[pl.ds(base, ROWS_PER_TILE)])

    return k(dst2d)


def _conv_kernel(table, src1d, dst2d, zeros2d):
    @functools.partial(
        pl.kernel, mesh=_MESH,
        out_type=jax.ShapeDtypeStruct((2, NPAD, 128), F32),
        scratch_types=[
            pltpu.VMEM((NJ, 128), jnp.int32),
            pltpu.VMEM((128,), jnp.int32),
            pltpu.VMEM((128,), jnp.int32),
            pltpu.VMEM((128, 128), F32),
            pltpu.VMEM((128, 128), F32),
            pltpu.VMEM_SHARED((NPAD, 128), F32),
            pltpu.SemaphoreType.DMA,
            pltpu.SemaphoreType.DMA,
            pltpu.SemaphoreType.DMA,
            pltpu.SemaphoreType.DMA,
            pltpu.SemaphoreType.DMA,
            pltpu.SemaphoreType.DMA,
            pltpu.SemaphoreType.DMA,
        ],
    )
    def k(tab_hbm, src_hbm, dst_hbm, z_hbm, out_hbm,
          dst_v, si0, si1, rows0, rows1, acc_sh,
          is0, is1, gs0, gs1, ss0, ss1, zs):
        cid = lax.axis_index("c")
        sid = lax.axis_index("s")
        base = sid * ROWS_PER_TILE
        ebase = (cid * 16 + sid) * NJ

        sidx = (si0, si1)
        rows = (rows0, rows1)
        isem = (is0, is1)
        gsem = (gs0, gs1)
        ssem = (ss0, ss1)

        def start_idx(j, b):
            return pltpu.async_copy(
                src_hbm.at[pl.ds((ebase + j) * 128, 128)], sidx[b], isem[b])

        def wait_idx(b):
            pltpu.make_async_copy(
                src_hbm.at[pl.ds(0, 128)], sidx[b], isem[b]).wait()

        def start_gather(b):
            return pltpu.async_copy(tab_hbm.at[sidx[b]], rows[b], gsem[b])

        def start_scatter(j, b):
            return pltpu.async_copy(rows[b], acc_sh.at[dst_v.at[j]],
                                    ssem[b], add=True)

        def wait_scatter(b):
            pltpu.make_async_copy(z_hbm, rows[b], ssem[b]).wait()

        pltpu.sync_copy(z_hbm, rows0)
        zc = []
        for t in range(ROWS_PER_TILE // 128):
            zc.append(pltpu.async_copy(
                rows0, acc_sh.at[pl.ds(base + t * 128, 128)], zs))
        pltpu.sync_copy(dst_hbm.at[pl.ds(ebase, NJ)], dst_v)
        for c in zc:
            c.wait()
        plsc.subcore_barrier()

        start_idx(0, 0)
        start_idx(1, 1)
        wait_idx(0)
        g0 = start_gather(0)
        wait_idx(1)
        g1 = start_gather(1)
        g0.wait()
        start_idx(2, 0)
        start_scatter(0, 0)
        g1.wait()
        start_idx(3, 1)
        start_scatter(1, 1)

        @pl.loop(1, NJ // 2 - 1)
        def _(g):
            j0 = 2 * g
            wait_scatter(0)
            wait_idx(0)
            ga = start_gather(0)
            wait_scatter(1)
            wait_idx(1)
            gb = start_gather(1)
            ga.wait()
            start_idx(j0 + 2, 0)
            start_scatter(j0, 0)
            gb.wait()
            start_idx(j0 + 3, 1)
            start_scatter(j0 + 1, 1)

        wait_scatter(0)
        wait_idx(0)
        g0 = start_gather(0)
        wait_scatter(1)
        wait_idx(1)
        g1 = start_gather(1)
        g0.wait()
        s0 = start_scatter(NJ - 2, 0)
        g1.wait()
        s1 = start_scatter(NJ - 1, 1)
        s0.wait()
        s1.wait()

        plsc.subcore_barrier()

        oc = []
        for t in range(ROWS_PER_TILE // 128):
            r0 = base + t * 128
            oc.append(pltpu.async_copy(
                acc_sh.at[pl.ds(r0, 128)], out_hbm.at[cid].at[pl.ds(r0, 128)],
                zs))
        for c in oc:
            c.wait()

    return k(table, src1d, dst2d, zeros2d)


def _ln(x, g, b, eps=1e-5):
    mu = jnp.mean(x, axis=-1, keepdims=True)
    var = jnp.mean(jnp.square(x - mu), axis=-1, keepdims=True)
    return (x - mu) * lax.rsqrt(var + eps) * g + b


def _elu(x):
    return jnp.where(x > 0, x, jnp.exp(x) - 1.0)


def _row_spec(f):
    return pl.BlockSpec((BN, f), lambda i: (i, 0))


def _full_spec(shape):
    return pl.BlockSpec(shape, lambda i: tuple(0 for _ in shape))


def _stack_spec():
    return pl.BlockSpec((2, BN, 128), lambda i: (0, i, 0))


def _tc1(deg0_col, deg1_col, x, W1):
    def body(d0_r, d1_r, x_r, w_r, dinv_r, p_r):
        dinv = lax.rsqrt(d0_r[...] + d1_r[...] + 1.0)
        dinv_r[...] = dinv
        p_r[...] = dinv * jnp.dot(x_r[...], w_r[...],
                                  preferred_element_type=F32)

    return pl.pallas_call(
        body,
        grid=(NPAD // BN,),
        in_specs=[_row_spec(1), _row_spec(1), _row_spec(128),
                  _full_spec((128, 128))],
        out_specs=[_row_spec(1), _row_spec(128)],
        out_shape=[jax.ShapeDtypeStruct((NPAD, 1), F32),
                   jax.ShapeDtypeStruct((NPAD, 128), F32)],
    )(deg0_col, deg1_col, x, W1)


def _tc2(dinv_col, s1, p1, b1, g1, be1):
    def body(dinv_r, s_r, p_r, b_r, g_r, be_r, o_r):
        dinv = dinv_r[...]
        pre = dinv * (s_r[0] + s_r[1] + p_r[...]) + b_r[...]
        h = _elu(_ln(pre, g_r[...], be_r[...]))
        o_r[...] = dinv * h

    return pl.pallas_call(
        body,
        grid=(NPAD // BN,),
        in_specs=[_row_spec(1), _stack_spec(), _row_spec(128),
                  _full_spec((1, 128)), _full_spec((1, 128)),
                  _full_spec((1, 128))],
        out_specs=_row_spec(128),
        out_shape=jax.ShapeDtypeStruct((NPAD, 128), F32),
    )(dinv_col, s1, p1, b1, g1, be1)


def _tc3(dinv_col, s2, p2, b2, g2, be2, W2, W3):
    def body(dinv_r, s_r, p_r, b_r, g_r, be_r, w2_r, w3_r, o_r):
        dinv = dinv_r[...]
        z = jnp.dot(s_r[0] + s_r[1] + p_r[...], w2_r[...],
                    preferred_element_type=F32)
        pre = dinv * z + b_r[...]
        h = _elu(_ln(pre, g_r[...], be_r[...]))
        o_r[...] = dinv * jnp.dot(h, w3_r[...], preferred_element_type=F32)

    return pl.pallas_call(
        body,
        grid=(NPAD // BN,),
        in_specs=[_row_spec(1), _stack_spec(), _row_spec(128),
                  _full_spec((1, 256)), _full_spec((1, 256)),
                  _full_spec((1, 256)), _full_spec((128, 256)),
                  _full_spec((256, 128))],
        out_specs=_row_spec(128),
        out_shape=jax.ShapeDtypeStruct((NPAD, 128), F32),
    )(dinv_col, s2, p2, b2, g2, be2, W2, W3)


def _tc4(dinv_col, s3, p3, b3, g3, be3, Wl1, bl1, g4, be4, Wl2, bl2):
    def body(dinv_r, s_r, p_r, b_r, g_r, be_r, w1_r, b1_r, g4_r, be4_r,
             w2_r, b2_r, o_r):
        dinv = dinv_r[...]
        pre = dinv * (s_r[0] + s_r[1] + p_r[...]) + b_r[...]
        h = _elu(_ln(pre, g_r[...], be_r[...]))
        h = jnp.dot(h, w1_r[...], preferred_element_type=F32) + b1_r[...]
        h = _elu(_ln(h, g4_r[...], be4_r[...]))
        o_r[...] = jnp.dot(h, w2_r[...], preferred_element_type=F32) + b2_r[...]

    return pl.pallas_call(
        body,
        grid=(NPAD // BN,),
        in_specs=[_row_spec(1), _stack_spec(), _row_spec(128),
                  _full_spec((1, 128)), _full_spec((1, 128)),
                  _full_spec((1, 128)), _full_spec((128, 64)),
                  _full_spec((1, 64)), _full_spec((1, 64)),
                  _full_spec((1, 64)), _full_spec((64, 64)),
                  _full_spec((1, 64))],
        out_specs=_row_spec(64),
        out_shape=jax.ShapeDtypeStruct((NPAD, 64), F32),
    )(dinv_col, s3, p3, b3, g3, be3, Wl1, bl1, g4, be4, Wl2, bl2)


def kernel(x, edge_index, W1, b1, g1, be1, W2, b2, g2, be2, W3, b3, g3, be3,
           Wl1, bl1, g4, be4, Wl2, bl2):
    pad = EPAD - E
    pad_idx = N + (jnp.arange(pad, dtype=jnp.int32) % 128)
    src1d = jnp.concatenate([edge_index[0], pad_idx])
    dst2d = jnp.concatenate([edge_index[1], pad_idx]).reshape(NROWS, 128)
    zeros2d = jnp.zeros((128, 128), F32)
    x_pad = jnp.pad(x, ((0, NPAD - N), (0, 0)))

    row = lambda v: v.reshape(1, -1)
    col = lambda v: v.reshape(NPAD, 1)

    deg0, deg1 = _deg_kernel(dst2d)
    dinv_col, p1 = _tc1(col(deg0), col(deg1), x_pad, W1)
    s1 = _conv_kernel(p1, src1d, dst2d, zeros2d)
    p2 = _tc2(dinv_col, s1, p1, row(b1), row(g1), row(be1))
    s2 = _conv_kernel(p2, src1d, dst2d, zeros2d)
    p3 = _tc3(dinv_col, s2, p2, row(b2), row(g2), row(be2), W2, W3)
    s3 = _conv_kernel(p3, src1d, dst2d, zeros2d)
    out = _tc4(dinv_col, s3, p3, row(b3), row(g3), row(be3),
               Wl1, row(bl1), row(g4), row(be4), Wl2, row(bl2))
    return out[:N]


# speedup vs baseline: 32.0567x; 1.3130x over previous
import functools

import jax
import jax.numpy as jnp
from jax import lax
from jax.experimental import pallas as pl
from jax.experimental.pallas import tpu as pltpu
from jax.experimental.pallas import tpu_sc as plsc

N = 10000
NPAD = 10240
E = 320000
EPAD = 327680
NROWS = EPAD // 128
NJ = NROWS // 32
ROWS_PER_TILE = NPAD // 16
BN = 1280
F32 = jnp.float32

_MESH = plsc.VectorSubcoreMesh(core_axis_name="c", subcore_axis_name="s")


def _deg_kernel(dst2d):
    @functools.partial(
        pl.kernel, mesh=_MESH,
        out_type=[jax.ShapeDtypeStruct((NPAD,), F32),
                  jax.ShapeDtypeStruct((NPAD,), F32)],
        scratch_types=[
            pltpu.VMEM((NJ, 128), jnp.int32),
            pltpu.VMEM((128,), F32),
            pltpu.VMEM((ROWS_PER_TILE,), F32),
            pltpu.VMEM_SHARED((NPAD,), F32),
            pltpu.SemaphoreType.DMA,
        ],
    )
    def k(dst_hbm, out0_hbm, out1_hbm, dst_v, ones_v, ztile_v, deg_sh, sem):
        cid = lax.axis_index("c")
        sid = lax.axis_index("s")
        base = sid * ROWS_PER_TILE
        erow = (cid * 16 + sid) * NJ

        @pl.loop(0, ROWS_PER_TILE // 16)
        def _(i):
            ztile_v[pl.ds(i * 16, 16)] = jnp.zeros((16,), F32)

        pltpu.sync_copy(ztile_v, deg_sh.at[pl.ds(base, ROWS_PER_TILE)])
        pltpu.sync_copy(dst_hbm.at[pl.ds(erow, NJ)], dst_v)

        @pl.loop(0, 8)
        def _(i):
            ones_v[pl.ds(i * 16, 16)] = jnp.ones((16,), F32)

        plsc.subcore_barrier()

        @pl.loop(0, NJ)
        def _(j):
            pltpu.sync_copy(ones_v, deg_sh.at[dst_v.at[j]], add=True)

        plsc.subcore_barrier()

        @pl.when(cid == 0)
        def _():
            pltpu.sync_copy(deg_sh.at[pl.ds(base, ROWS_PER_TILE)], ztile_v)
            pltpu.sync_copy(ztile_v, out0_hbm.at[pl.ds(base, ROWS_PER_TILE)])

        @pl.when(cid == 1)
        def _():
            pltpu.sync_copy(deg_sh.at[pl.ds(base, ROWS_PER_TILE)], ztile_v)
            pltpu.sync_copy(ztile_v, out1_hbm.at[pl.ds(base, ROWS_PER_TILE)])

    return k(dst2d)


def _conv_kernel(table, src1d, dst1d, zeros2d):
    C = 64
    NB = 4
    NJC = EPAD // C // 32

    @functools.partial(
        pl.kernel, mesh=_MESH,
        out_type=jax.ShapeDtypeStruct((2, NPAD, 128), F32),
        scratch_types=[
            pltpu.VMEM((C,), jnp.int32),
            pltpu.VMEM((C,), jnp.int32),
            pltpu.VMEM((C,), jnp.int32),
            pltpu.VMEM((C,), jnp.int32),
            pltpu.VMEM((C,), jnp.int32),
            pltpu.VMEM((C,), jnp.int32),
            pltpu.VMEM((C,), jnp.int32),
            pltpu.VMEM((C,), jnp.int32),
            pltpu.VMEM((C, 128), F32),
            pltpu.VMEM((C, 128), F32),
            pltpu.VMEM((C, 128), F32),
            pltpu.VMEM((C, 128), F32),
            pltpu.VMEM_SHARED((NPAD, 128), F32),
            pltpu.SemaphoreType.DMA,
            pltpu.SemaphoreType.DMA,
            pltpu.SemaphoreType.DMA,
            pltpu.SemaphoreType.DMA,
            pltpu.SemaphoreType.DMA,
            pltpu.SemaphoreType.DMA,
            pltpu.SemaphoreType.DMA,
            pltpu.SemaphoreType.DMA,
            pltpu.SemaphoreType.DMA,
            pltpu.SemaphoreType.DMA,
            pltpu.SemaphoreType.DMA,
            pltpu.SemaphoreType.DMA,
            pltpu.SemaphoreType.DMA,
        ],
    )
    def k(tab_hbm, src_hbm, dst_hbm, z_hbm, out_hbm,
          si0, si1, si2, si3, di0, di1, di2, di3, r0, r1, r2, r3, acc_sh,
          ia, ib, ic, id_, ga, gb, gc, gd, sa, sb, sc, sd, zs):
        cid = lax.axis_index("c")
        sid = lax.axis_index("s")
        base = sid * ROWS_PER_TILE
        ebase = (cid * 16 + sid) * NJC

        sidx = (si0, si1, si2, si3)
        didx = (di0, di1, di2, di3)
        rows = (r0, r1, r2, r3)
        isem = (ia, ib, ic, id_)
        gsem = (ga, gb, gc, gd)
        ssem = (sa, sb, sc, sd)

        def start_idx(j, b):
            off = (ebase + j) * C
            pltpu.async_copy(src_hbm.at[pl.ds(off, C)], sidx[b], isem[b])
            pltpu.async_copy(dst_hbm.at[pl.ds(off, C)], didx[b], isem[b])

        def wait_idx(b):
            pltpu.make_async_copy(src_hbm.at[pl.ds(0, C)], sidx[b],
                                  isem[b]).wait()
            pltpu.make_async_copy(dst_hbm.at[pl.ds(0, C)], didx[b],
                                  isem[b]).wait()

        def start_gather(b):
            pltpu.async_copy(tab_hbm.at[sidx[b]], rows[b], gsem[b])

        def wait_gather(b):
            pltpu.make_async_copy(tab_hbm.at[sidx[b]], rows[b],
                                  gsem[b]).wait()

        def start_scatter(b):
            pltpu.async_copy(rows[b], acc_sh.at[didx[b]], ssem[b], add=True)

        def wait_scatter(b):
            pltpu.make_async_copy(z_hbm.at[pl.ds(0, C)], rows[b],
                                  ssem[b]).wait()

        def slot(j, jj):
            wait_scatter((jj + NB - 1) % NB)
            start_idx(j + 3, (jj + 3) % NB)
            wait_idx((jj + 2) % NB)
            start_gather((jj + 2) % NB)
            wait_gather(jj)
            start_scatter(jj)

        pltpu.sync_copy(z_hbm.at[pl.ds(0, C)], r0)
        zc = []
        for t in range(ROWS_PER_TILE // C):
            zc.append(pltpu.async_copy(
                r0, acc_sh.at[pl.ds(base + t * C, C)], zs))
        for c in zc:
            c.wait()
        plsc.subcore_barrier()

        start_idx(0, 0)
        start_idx(1, 1)
        start_idx(2, 2)
        wait_idx(0)
        start_gather(0)
        wait_idx(1)
        start_gather(1)
        start_idx(3, 3)
        wait_idx(2)
        start_gather(2)
        wait_gather(0)
        start_scatter(0)
        slot(1, 1)
        slot(2, 2)
        slot(3, 3)

        @pl.loop(1, NJC // NB - 1)
        def _(g):
            j0 = NB * g
            slot(j0, 0)
            slot(j0 + 1, 1)
            slot(j0 + 2, 2)
            slot(j0 + 3, 3)

        wait_scatter(3)
        start_idx(NJC - 1, 3)
        wait_idx(2)
        start_gather(2)
        wait_gather(0)
        start_scatter(0)

        wait_scatter(0)
        wait_idx(3)
        start_gather(3)
        wait_gather(1)
        start_scatter(1)

        wait_scatter(1)
        wait_gather(2)
        start_scatter(2)

        wait_scatter(2)
        wait_gather(3)
        start_scatter(3)
        wait_scatter(3)

        plsc.subcore_barrier()

        oc = []
        for t in range(ROWS_PER_TILE // 128):
            rr = base + t * 128
            oc.append(pltpu.async_copy(
                acc_sh.at[pl.ds(rr, 128)], out_hbm.at[cid].at[pl.ds(rr, 128)],
                zs))
        for c in oc:
            c.wait()

    return k(table, src1d, dst1d, zeros2d)


def _ln(x, g, b, eps=1e-5):
    mu = jnp.mean(x, axis=-1, keepdims=True)
    var = jnp.mean(jnp.square(x - mu), axis=-1, keepdims=True)
    return (x - mu) * lax.rsqrt(var + eps) * g + b


def _elu(x):
    return jnp.where(x > 0, x, jnp.exp(x) - 1.0)


def _row_spec(f):
    return pl.BlockSpec((BN, f), lambda i: (i, 0))


def _full_spec(shape):
    return pl.BlockSpec(shape, lambda i: tuple(0 for _ in shape))


def _stack_spec():
    return pl.BlockSpec((2, BN, 128), lambda i: (0, i, 0))


def _tc1(deg0_col, deg1_col, x, W1):
    def body(d0_r, d1_r, x_r, w_r, dinv_r, p_r):
        dinv = lax.rsqrt(d0_r[...] + d1_r[...] + 1.0)
        dinv_r[...] = dinv
        p_r[...] = dinv * jnp.dot(x_r[...], w_r[...],
                                  preferred_element_type=F32)

    return pl.pallas_call(
        body,
        grid=(NPAD // BN,),
        in_specs=[_row_spec(1), _row_spec(1), _row_spec(128),
                  _full_spec((128, 128))],
        out_specs=[_row_spec(1), _row_spec(128)],
        out_shape=[jax.ShapeDtypeStruct((NPAD, 1), F32),
                   jax.ShapeDtypeStruct((NPAD, 128), F32)],
    )(deg0_col, deg1_col, x, W1)


def _tc2(dinv_col, s1, p1, b1, g1, be1):
    def body(dinv_r, s_r, p_r, b_r, g_r, be_r, o_r):
        dinv = dinv_r[...]
        pre = dinv * (s_r[0] + s_r[1] + p_r[...]) + b_r[...]
        h = _elu(_ln(pre, g_r[...], be_r[...]))
        o_r[...] = dinv * h

    return pl.pallas_call(
        body,
        grid=(NPAD // BN,),
        in_specs=[_row_spec(1), _stack_spec(), _row_spec(128),
                  _full_spec((1, 128)), _full_spec((1, 128)),
                  _full_spec((1, 128))],
        out_specs=_row_spec(128),
        out_shape=jax.ShapeDtypeStruct((NPAD, 128), F32),
    )(dinv_col, s1, p1, b1, g1, be1)


def _tc3(dinv_col, s2, p2, b2, g2, be2, W2, W3):
    def body(dinv_r, s_r, p_r, b_r, g_r, be_r, w2_r, w3_r, o_r):
        dinv = dinv_r[...]
        z = jnp.dot(s_r[0] + s_r[1] + p_r[...], w2_r[...],
                    preferred_element_type=F32)
        pre = dinv * z + b_r[...]
        h = _elu(_ln(pre, g_r[...], be_r[...]))
        o_r[...] = dinv * jnp.dot(h, w3_r[...], preferred_element_type=F32)

    return pl.pallas_call(
        body,
        grid=(NPAD // BN,),
        in_specs=[_row_spec(1), _stack_spec(), _row_spec(128),
                  _full_spec((1, 256)), _full_spec((1, 256)),
                  _full_spec((1, 256)), _full_spec((128, 256)),
                  _full_spec((256, 128))],
        out_specs=_row_spec(128),
        out_shape=jax.ShapeDtypeStruct((NPAD, 128), F32),
    )(dinv_col, s2, p2, b2, g2, be2, W2, W3)


def _tc4(dinv_col, s3, p3, b3, g3, be3, Wl1, bl1, g4, be4, Wl2, bl2):
    def body(dinv_r, s_r, p_r, b_r, g_r, be_r, w1_r, b1_r, g4_r, be4_r,
             w2_r, b2_r, o_r):
        dinv = dinv_r[...]
        pre = dinv * (s_r[0] + s_r[1] + p_r[...]) + b_r[...]
        h = _elu(_ln(pre, g_r[...], be_r[...]))
        h = jnp.dot(h, w1_r[...], preferred_element_type=F32) + b1_r[...]
        h = _elu(_ln(h, g4_r[...], be4_r[...]))
        o_r[...] = jnp.dot(h, w2_r[...], preferred_element_type=F32) + b2_r[...]

    return pl.pallas_call(
        body,
        grid=(NPAD // BN,),
        in_specs=[_row_spec(1), _stack_spec(), _row_spec(128),
                  _full_spec((1, 128)), _full_spec((1, 128)),
                  _full_spec((1, 128)), _full_spec((128, 64)),
                  _full_spec((1, 64)), _full_spec((1, 64)),
                  _full_spec((1, 64)), _full_spec((64, 64)),
                  _full_spec((1, 64))],
        out_specs=_row_spec(64),
        out_shape=jax.ShapeDtypeStruct((NPAD, 64), F32),
    )(dinv_col, s3, p3, b3, g3, be3, Wl1, bl1, g4, be4, Wl2, bl2)


def kernel(x, edge_index, W1, b1, g1, be1, W2, b2, g2, be2, W3, b3, g3, be3,
           Wl1, bl1, g4, be4, Wl2, bl2):
    pad = EPAD - E
    pad_idx = N + (jnp.arange(pad, dtype=jnp.int32) % 128)
    src1d = jnp.concatenate([edge_index[0], pad_idx])
    dst1d = jnp.concatenate([edge_index[1], pad_idx])
    dst2d = dst1d.reshape(NROWS, 128)
    zeros2d = jnp.zeros((128, 128), F32)
    x_pad = jnp.pad(x, ((0, NPAD - N), (0, 0)))

    row = lambda v: v.reshape(1, -1)
    col = lambda v: v.reshape(NPAD, 1)

    deg0, deg1 = _deg_kernel(dst2d)
    dinv_col, p1 = _tc1(col(deg0), col(deg1), x_pad, W1)
    s1 = _conv_kernel(p1, src1d, dst1d, zeros2d)
    p2 = _tc2(dinv_col, s1, p1, row(b1), row(g1), row(be1))
    s2 = _conv_kernel(p2, src1d, dst1d, zeros2d)
    p3 = _tc3(dinv_col, s2, p2, row(b2), row(g2), row(be2), W2, W3)
    s3 = _conv_kernel(p3, src1d, dst1d, zeros2d)
    out = _tc4(dinv_col, s3, p3, row(b3), row(g3), row(be3),
               Wl1, row(bl1), row(g4), row(be4), Wl2, row(bl2))
    return out[:N]


# speedup vs baseline: 33.5018x; 1.0451x over previous
import functools

import jax
import jax.numpy as jnp
from jax import lax
from jax.experimental import pallas as pl
from jax.experimental.pallas import tpu as pltpu
from jax.experimental.pallas import tpu_sc as plsc

N = 10000
NPAD = 10240
E = 320000
EPAD = 327680
NROWS = EPAD // 128
NJ = NROWS // 32
ROWS_PER_TILE = NPAD // 16
BN = 1280
F32 = jnp.float32

_MESH = plsc.VectorSubcoreMesh(core_axis_name="c", subcore_axis_name="s")


def _deg_kernel(dst2d):
    @functools.partial(
        pl.kernel, mesh=_MESH,
        out_type=[jax.ShapeDtypeStruct((NPAD,), F32),
                  jax.ShapeDtypeStruct((NPAD,), F32)],
        scratch_types=[
            pltpu.VMEM((NJ, 128), jnp.int32),
            pltpu.VMEM((128,), F32),
            pltpu.VMEM((ROWS_PER_TILE,), F32),
            pltpu.VMEM_SHARED((NPAD,), F32),
            pltpu.SemaphoreType.DMA,
        ],
    )
    def k(dst_hbm, out0_hbm, out1_hbm, dst_v, ones_v, ztile_v, deg_sh, sem):
        cid = lax.axis_index("c")
        sid = lax.axis_index("s")
        base = sid * ROWS_PER_TILE
        erow = (cid * 16 + sid) * NJ

        @pl.loop(0, ROWS_PER_TILE // 16)
        def _(i):
            ztile_v[pl.ds(i * 16, 16)] = jnp.zeros((16,), F32)

        pltpu.sync_copy(ztile_v, deg_sh.at[pl.ds(base, ROWS_PER_TILE)])
        pltpu.sync_copy(dst_hbm.at[pl.ds(erow, NJ)], dst_v)

        @pl.loop(0, 8)
        def _(i):
            ones_v[pl.ds(i * 16, 16)] = jnp.ones((16,), F32)

        plsc.subcore_barrier()

        @pl.loop(0, NJ)
        def _(j):
            pltpu.sync_copy(ones_v, deg_sh.at[dst_v.at[j]], add=True)

        plsc.subcore_barrier()

        @pl.when(cid == 0)
        def _():
            pltpu.sync_copy(deg_sh.at[pl.ds(base, ROWS_PER_TILE)], ztile_v)
            pltpu.sync_copy(ztile_v, out0_hbm.at[pl.ds(base, ROWS_PER_TILE)])

        @pl.when(cid == 1)
        def _():
            pltpu.sync_copy(deg_sh.at[pl.ds(base, ROWS_PER_TILE)], ztile_v)
            pltpu.sync_copy(ztile_v, out1_hbm.at[pl.ds(base, ROWS_PER_TILE)])

    return k(dst2d)


def _conv_kernel(table, src1d, dst1d, zeros2d):
    C = 64
    NB = 5
    NJC = EPAD // C // 32

    @functools.partial(
        pl.kernel, mesh=_MESH,
        out_type=jax.ShapeDtypeStruct((2, NPAD, 128), F32),
        scratch_types=(
            [pltpu.VMEM((C,), jnp.int32)] * NB +
            [pltpu.VMEM((C,), jnp.int32)] * NB +
            [pltpu.VMEM((C, 128), F32)] * NB +
            [pltpu.VMEM_SHARED((NPAD, 128), F32)] +
            [pltpu.SemaphoreType.DMA] * (3 * NB + 1)
        ),
    )
    def k(tab_hbm, src_hbm, dst_hbm, z_hbm, out_hbm, *bufs):
        sidx = bufs[0:NB]
        didx = bufs[NB:2 * NB]
        rows = bufs[2 * NB:3 * NB]
        acc_sh = bufs[3 * NB]
        isem = bufs[3 * NB + 1:4 * NB + 1]
        gsem = bufs[4 * NB + 1:5 * NB + 1]
        ssem = bufs[5 * NB + 1:6 * NB + 1]
        zs = bufs[6 * NB + 1]
        r0 = rows[0]
        cid = lax.axis_index("c")
        sid = lax.axis_index("s")
        base = sid * ROWS_PER_TILE
        ebase = (cid * 16 + sid) * NJC

        def start_idx(j, b):
            off = (ebase + j) * C
            pltpu.async_copy(src_hbm.at[pl.ds(off, C)], sidx[b], isem[b])
            pltpu.async_copy(dst_hbm.at[pl.ds(off, C)], didx[b], isem[b])

        def wait_idx(b):
            pltpu.make_async_copy(src_hbm.at[pl.ds(0, C)], sidx[b],
                                  isem[b]).wait()
            pltpu.make_async_copy(dst_hbm.at[pl.ds(0, C)], didx[b],
                                  isem[b]).wait()

        def start_gather(b):
            pltpu.async_copy(tab_hbm.at[sidx[b]], rows[b], gsem[b])

        def wait_gather(b):
            pltpu.make_async_copy(tab_hbm.at[sidx[b]], rows[b],
                                  gsem[b]).wait()

        def start_scatter(b):
            pltpu.async_copy(rows[b], acc_sh.at[didx[b]], ssem[b], add=True)

        def wait_scatter(b):
            pltpu.make_async_copy(z_hbm.at[pl.ds(0, C)], rows[b],
                                  ssem[b]).wait()

        def slot(j, jj):
            wait_scatter((jj + NB - 1) % NB)
            start_idx(j + 4, (jj + 4) % NB)
            wait_idx((jj + 3) % NB)
            start_gather((jj + 3) % NB)
            wait_gather(jj)
            start_scatter(jj)

        pltpu.sync_copy(z_hbm.at[pl.ds(0, C)], r0)
        zc = []
        for t in range(ROWS_PER_TILE // C):
            zc.append(pltpu.async_copy(
                r0, acc_sh.at[pl.ds(base + t * C, C)], zs))
        for c in zc:
            c.wait()
        plsc.subcore_barrier()

        start_idx(0, 0)
        start_idx(1, 1)
        start_idx(2, 2)
        start_idx(3, 3)
        wait_idx(0)
        start_gather(0)
        wait_idx(1)
        start_gather(1)
        wait_idx(2)
        start_gather(2)
        start_idx(4, 4)
        wait_idx(3)
        start_gather(3)
        wait_gather(0)
        start_scatter(0)
        slot(1, 1)
        slot(2, 2)
        slot(3, 3)
        slot(4, 4)

        @pl.loop(1, NJC // NB - 1)
        def _(g):
            j0 = NB * g
            slot(j0, 0)
            slot(j0 + 1, 1)
            slot(j0 + 2, 2)
            slot(j0 + 3, 3)
            slot(j0 + 4, 4)

        wait_scatter(4)
        start_idx(NJC - 1, 4)
        wait_idx(3)
        start_gather(3)
        wait_gather(0)
        start_scatter(0)

        wait_scatter(0)
        wait_idx(4)
        start_gather(4)
        wait_gather(1)
        start_scatter(1)

        wait_scatter(1)
        wait_gather(2)
        start_scatter(2)

        wait_scatter(2)
        wait_gather(3)
        start_scatter(3)

        wait_scatter(3)
        wait_gather(4)
        start_scatter(4)
        wait_scatter(4)

        plsc.subcore_barrier()

        oc = []
        for t in range(ROWS_PER_TILE // 128):
            rr = base + t * 128
            oc.append(pltpu.async_copy(
                acc_sh.at[pl.ds(rr, 128)], out_hbm.at[cid].at[pl.ds(rr, 128)],
                zs))
        for c in oc:
            c.wait()

    return k(table, src1d, dst1d, zeros2d)


def _ln(x, g, b, eps=1e-5):
    mu = jnp.mean(x, axis=-1, keepdims=True)
    var = jnp.mean(jnp.square(x - mu), axis=-1, keepdims=True)
    return (x - mu) * lax.rsqrt(var + eps) * g + b


def _elu(x):
    return jnp.where(x > 0, x, jnp.exp(x) - 1.0)


def _row_spec(f):
    return pl.BlockSpec((BN, f), lambda i: (i, 0))


def _full_spec(shape):
    return pl.BlockSpec(shape, lambda i: tuple(0 for _ in shape))


def _stack_spec():
    return pl.BlockSpec((2, BN, 128), lambda i: (0, i, 0))


def _tc1(deg0_col, deg1_col, x, W1):
    def body(d0_r, d1_r, x_r, w_r, dinv_r, p_r):
        dinv = lax.rsqrt(d0_r[...] + d1_r[...] + 1.0)
        dinv_r[...] = dinv
        p_r[...] = dinv * jnp.dot(x_r[...], w_r[...],
                                  preferred_element_type=F32)

    return pl.pallas_call(
        body,
        grid=(NPAD // BN,),
        in_specs=[_row_spec(1), _row_spec(1), _row_spec(128),
                  _full_spec((128, 128))],
        out_specs=[_row_spec(1), _row_spec(128)],
        out_shape=[jax.ShapeDtypeStruct((NPAD, 1), F32),
                   jax.ShapeDtypeStruct((NPAD, 128), F32)],
    )(deg0_col, deg1_col, x, W1)


def _tc2(dinv_col, s1, p1, b1, g1, be1):
    def body(dinv_r, s_r, p_r, b_r, g_r, be_r, o_r):
        dinv = dinv_r[...]
        pre = dinv * (s_r[0] + s_r[1] + p_r[...]) + b_r[...]
        h = _elu(_ln(pre, g_r[...], be_r[...]))
        o_r[...] = dinv * h

    return pl.pallas_call(
        body,
        grid=(NPAD // BN,),
        in_specs=[_row_spec(1), _stack_spec(), _row_spec(128),
                  _full_spec((1, 128)), _full_spec((1, 128)),
                  _full_spec((1, 128))],
        out_specs=_row_spec(128),
        out_shape=jax.ShapeDtypeStruct((NPAD, 128), F32),
    )(dinv_col, s1, p1, b1, g1, be1)


def _tc3(dinv_col, s2, p2, b2, g2, be2, W2, W3):
    def body(dinv_r, s_r, p_r, b_r, g_r, be_r, w2_r, w3_r, o_r):
        dinv = dinv_r[...]
        z = jnp.dot(s_r[0] + s_r[1] + p_r[...], w2_r[...],
                    preferred_element_type=F32)
        pre = dinv * z + b_r[...]
        h = _elu(_ln(pre, g_r[...], be_r[...]))
        o_r[...] = dinv * jnp.dot(h, w3_r[...], preferred_element_type=F32)

    return pl.pallas_call(
        body,
        grid=(NPAD // BN,),
        in_specs=[_row_spec(1), _stack_spec(), _row_spec(128),
                  _full_spec((1, 256)), _full_spec((1, 256)),
                  _full_spec((1, 256)), _full_spec((128, 256)),
                  _full_spec((256, 128))],
        out_specs=_row_spec(128),
        out_shape=jax.ShapeDtypeStruct((NPAD, 128), F32),
    )(dinv_col, s2, p2, b2, g2, be2, W2, W3)


def _tc4(dinv_col, s3, p3, b3, g3, be3, Wl1, bl1, g4, be4, Wl2, bl2):
    def body(dinv_r, s_r, p_r, b_r, g_r, be_r, w1_r, b1_r, g4_r, be4_r,
             w2_r, b2_r, o_r):
        dinv = dinv_r[...]
        pre = dinv * (s_r[0] + s_r[1] + p_r[...]) + b_r[...]
        h = _elu(_ln(pre, g_r[...], be_r[...]))
        h = jnp.dot(h, w1_r[...], preferred_element_type=F32) + b1_r[...]
        h = _elu(_ln(h, g4_r[...], be4_r[...]))
        o_r[...] = jnp.dot(h, w2_r[...], preferred_element_type=F32) + b2_r[...]

    return pl.pallas_call(
        body,
        grid=(NPAD // BN,),
        in_specs=[_row_spec(1), _stack_spec(), _row_spec(128),
                  _full_spec((1, 128)), _full_spec((1, 128)),
                  _full_spec((1, 128)), _full_spec((128, 64)),
                  _full_spec((1, 64)), _full_spec((1, 64)),
                  _full_spec((1, 64)), _full_spec((64, 64)),
                  _full_spec((1, 64))],
        out_specs=_row_spec(64),
        out_shape=jax.ShapeDtypeStruct((NPAD, 64), F32),
    )(dinv_col, s3, p3, b3, g3, be3, Wl1, bl1, g4, be4, Wl2, bl2)


def kernel(x, edge_index, W1, b1, g1, be1, W2, b2, g2, be2, W3, b3, g3, be3,
           Wl1, bl1, g4, be4, Wl2, bl2):
    pad = EPAD - E
    pad_idx = N + (jnp.arange(pad, dtype=jnp.int32) % 128)
    src1d = jnp.concatenate([edge_index[0], pad_idx])
    dst1d = jnp.concatenate([edge_index[1], pad_idx])
    dst2d = dst1d.reshape(NROWS, 128)
    zeros2d = jnp.zeros((128, 128), F32)
    x_pad = jnp.pad(x, ((0, NPAD - N), (0, 0)))

    row = lambda v: v.reshape(1, -1)
    col = lambda v: v.reshape(NPAD, 1)

    deg0, deg1 = _deg_kernel(dst2d)
    dinv_col, p1 = _tc1(col(deg0), col(deg1), x_pad, W1)
    s1 = _conv_kernel(p1, src1d, dst1d, zeros2d)
    p2 = _tc2(dinv_col, s1, p1, row(b1), row(g1), row(be1))
    s2 = _conv_kernel(p2, src1d, dst1d, zeros2d)
    p3 = _tc3(dinv_col, s2, p2, row(b2), row(g2), row(be2), W2, W3)
    s3 = _conv_kernel(p3, src1d, dst1d, zeros2d)
    out = _tc4(dinv_col, s3, p3, row(b3), row(g3), row(be3),
               Wl1, row(bl1), row(g4), row(be4), Wl2, row(bl2))
    return out[:N]


# speedup vs baseline: 33.7366x; 1.0070x over previous
import functools

import jax
import jax.numpy as jnp
from jax import lax
from jax.experimental import pallas as pl
from jax.experimental.pallas import tpu as pltpu
from jax.experimental.pallas import tpu_sc as plsc

N = 10000
NPAD = 10240
E = 320000
EPAD = 327680
NROWS = EPAD // 128
NJ = NROWS // 32
ROWS_PER_TILE = NPAD // 16
BN = 1280
F32 = jnp.float32

_MESH = plsc.VectorSubcoreMesh(core_axis_name="c", subcore_axis_name="s")


def _deg_kernel(dst2d):
    @functools.partial(
        pl.kernel, mesh=_MESH,
        out_type=[jax.ShapeDtypeStruct((NPAD,), F32),
                  jax.ShapeDtypeStruct((NPAD,), F32)],
        scratch_types=[
            pltpu.VMEM((NJ, 128), jnp.int32),
            pltpu.VMEM((128,), F32),
            pltpu.VMEM((ROWS_PER_TILE,), F32),
            pltpu.VMEM_SHARED((NPAD,), F32),
            pltpu.SemaphoreType.DMA,
        ],
    )
    def k(dst_hbm, out0_hbm, out1_hbm, dst_v, ones_v, ztile_v, deg_sh, sem):
        cid = lax.axis_index("c")
        sid = lax.axis_index("s")
        base = sid * ROWS_PER_TILE
        erow = (cid * 16 + sid) * NJ

        @pl.loop(0, ROWS_PER_TILE // 16)
        def _(i):
            ztile_v[pl.ds(i * 16, 16)] = jnp.zeros((16,), F32)

        pltpu.sync_copy(ztile_v, deg_sh.at[pl.ds(base, ROWS_PER_TILE)])
        pltpu.sync_copy(dst_hbm.at[pl.ds(erow, NJ)], dst_v)

        @pl.loop(0, 8)
        def _(i):
            ones_v[pl.ds(i * 16, 16)] = jnp.ones((16,), F32)

        plsc.subcore_barrier()

        @pl.loop(0, NJ)
        def _(j):
            pltpu.sync_copy(ones_v, deg_sh.at[dst_v.at[j]], add=True)

        plsc.subcore_barrier()

        @pl.when(cid == 0)
        def _():
            pltpu.sync_copy(deg_sh.at[pl.ds(base, ROWS_PER_TILE)], ztile_v)
            pltpu.sync_copy(ztile_v, out0_hbm.at[pl.ds(base, ROWS_PER_TILE)])

        @pl.when(cid == 1)
        def _():
            pltpu.sync_copy(deg_sh.at[pl.ds(base, ROWS_PER_TILE)], ztile_v)
            pltpu.sync_copy(ztile_v, out1_hbm.at[pl.ds(base, ROWS_PER_TILE)])

    return k(dst2d)


def _conv_kernel(table, src1d, dst1d, zeros2d):
    C = 64
    NB = 5
    NJC = EPAD // C // 32

    @functools.partial(
        pl.kernel, mesh=_MESH,
        out_type=jax.ShapeDtypeStruct((2, NPAD, 128), F32),
        scratch_types=(
            [pltpu.VMEM((C,), jnp.int32)] * NB +
            [pltpu.VMEM((C,), jnp.int32)] * NB +
            [pltpu.VMEM((C, 128), F32)] * NB +
            [pltpu.VMEM_SHARED((NPAD, 128), F32)] +
            [pltpu.SemaphoreType.DMA] * (3 * NB + 1)
        ),
    )
    def k(tab_hbm, src_hbm, dst_hbm, z_hbm, out_hbm, *bufs):
        sidx = bufs[0:NB]
        didx = bufs[NB:2 * NB]
        rows = bufs[2 * NB:3 * NB]
        acc_sh = bufs[3 * NB]
        isem = bufs[3 * NB + 1:4 * NB + 1]
        gsem = bufs[4 * NB + 1:5 * NB + 1]
        ssem = bufs[5 * NB + 1:6 * NB + 1]
        zs = bufs[6 * NB + 1]
        r0 = rows[0]
        cid = lax.axis_index("c")
        sid = lax.axis_index("s")
        base = sid * ROWS_PER_TILE
        ebase = (cid * 16 + sid) * NJC

        def start_idx(j, b):
            off = (ebase + j) * C
            pltpu.async_copy(src_hbm.at[pl.ds(off, C)], sidx[b], isem[b])
            pltpu.async_copy(dst_hbm.at[pl.ds(off, C)], didx[b], isem[b])

        def wait_idx(b):
            pltpu.make_async_copy(src_hbm.at[pl.ds(0, C)], sidx[b],
                                  isem[b]).wait()
            pltpu.make_async_copy(dst_hbm.at[pl.ds(0, C)], didx[b],
                                  isem[b]).wait()

        def start_gather(b):
            pltpu.async_copy(tab_hbm.at[sidx[b]], rows[b], gsem[b])

        def wait_gather(b):
            pltpu.make_async_copy(tab_hbm.at[sidx[b]], rows[b],
                                  gsem[b]).wait()

        def start_scatter(b):
            pltpu.async_copy(rows[b], acc_sh.at[didx[b]], ssem[b], add=True)

        def wait_scatter(b):
            pltpu.make_async_copy(z_hbm.at[pl.ds(0, C)], rows[b],
                                  ssem[b]).wait()

        def slot(j, jj):
            wait_scatter((jj + NB - 1) % NB)
            start_idx(j + 4, (jj + 4) % NB)
            wait_idx((jj + 3) % NB)
            start_gather((jj + 3) % NB)
            wait_gather(jj)
            start_scatter(jj)

        pltpu.sync_copy(z_hbm.at[pl.ds(0, C)], r0)
        zc = []
        for t in range(ROWS_PER_TILE // C):
            zc.append(pltpu.async_copy(
                r0, acc_sh.at[pl.ds(base + t * C, C)], zs))
        for c in zc:
            c.wait()
        plsc.subcore_barrier()

        start_idx(0, 0)
        start_idx(1, 1)
        start_idx(2, 2)
        start_idx(3, 3)
        wait_idx(0)
        start_gather(0)
        wait_idx(1)
        start_gather(1)
        wait_idx(2)
        start_gather(2)
        start_idx(4, 4)
        wait_idx(3)
        start_gather(3)
        wait_gather(0)
        start_scatter(0)
        slot(1, 1)
        slot(2, 2)
        slot(3, 3)
        slot(4, 4)

        @pl.loop(1, NJC // NB - 1)
        def _(g):
            j0 = NB * g
            slot(j0, 0)
            slot(j0 + 1, 1)
            slot(j0 + 2, 2)
            slot(j0 + 3, 3)
            slot(j0 + 4, 4)

        wait_scatter(4)
        start_idx(NJC - 1, 4)
        wait_idx(3)
        start_gather(3)
        wait_gather(0)
        start_scatter(0)

        wait_scatter(0)
        wait_idx(4)
        start_gather(4)
        wait_gather(1)
        start_scatter(1)

        wait_scatter(1)
        wait_gather(2)
        start_scatter(2)

        wait_scatter(2)
        wait_gather(3)
        start_scatter(3)

        wait_scatter(3)
        wait_gather(4)
        start_scatter(4)
        wait_scatter(4)

        plsc.subcore_barrier()

        oc = []
        for t in range(ROWS_PER_TILE // 128):
            rr = base + t * 128
            oc.append(pltpu.async_copy(
                acc_sh.at[pl.ds(rr, 128)], out_hbm.at[cid].at[pl.ds(rr, 128)],
                zs))
        for c in oc:
            c.wait()

    return k(table, src1d, dst1d, zeros2d)


def _ln(x, g, b, eps=1e-5):
    mu = jnp.mean(x, axis=-1, keepdims=True)
    var = jnp.mean(jnp.square(x - mu), axis=-1, keepdims=True)
    return (x - mu) * lax.rsqrt(var + eps) * g + b


def _elu(x):
    return jnp.where(x > 0, x, jnp.exp(x) - 1.0)


def _row_spec(f):
    return pl.BlockSpec((BN, f), lambda i: (i, 0))


def _full_spec(shape):
    return pl.BlockSpec(shape, lambda i: tuple(0 for _ in shape))


def _stack_spec():
    return pl.BlockSpec((2, BN, 128), lambda i: (0, i, 0))


def _tc_mm(x, W1):
    def body(x_r, w_r, t_r):
        t_r[...] = jnp.dot(x_r[...], w_r[...], preferred_element_type=F32)

    return pl.pallas_call(
        body,
        grid=(NPAD // BN,),
        in_specs=[_row_spec(128), _full_spec((128, 128))],
        out_specs=_row_spec(128),
        out_shape=jax.ShapeDtypeStruct((NPAD, 128), F32),
    )(x, W1)


def _tc1(deg0_col, deg1_col, t1):
    def body(d0_r, d1_r, t_r, dinv_r, p_r):
        dinv = lax.rsqrt(d0_r[...] + d1_r[...] + 1.0)
        dinv_r[...] = dinv
        p_r[...] = dinv * t_r[...]

    return pl.pallas_call(
        body,
        grid=(NPAD // BN,),
        in_specs=[_row_spec(1), _row_spec(1), _row_spec(128)],
        out_specs=[_row_spec(1), _row_spec(128)],
        out_shape=[jax.ShapeDtypeStruct((NPAD, 1), F32),
                   jax.ShapeDtypeStruct((NPAD, 128), F32)],
    )(deg0_col, deg1_col, t1)


def _tc2(dinv_col, s1, p1, b1, g1, be1):
    def body(dinv_r, s_r, p_r, b_r, g_r, be_r, o_r):
        dinv = dinv_r[...]
        pre = dinv * (s_r[0] + s_r[1] + p_r[...]) + b_r[...]
        h = _elu(_ln(pre, g_r[...], be_r[...]))
        o_r[...] = dinv * h

    return pl.pallas_call(
        body,
        grid=(NPAD // BN,),
        in_specs=[_row_spec(1), _stack_spec(), _row_spec(128),
                  _full_spec((1, 128)), _full_spec((1, 128)),
                  _full_spec((1, 128))],
        out_specs=_row_spec(128),
        out_shape=jax.ShapeDtypeStruct((NPAD, 128), F32),
    )(dinv_col, s1, p1, b1, g1, be1)


def _tc3(dinv_col, s2, p2, b2, g2, be2, W2, W3):
    def body(dinv_r, s_r, p_r, b_r, g_r, be_r, w2_r, w3_r, o_r):
        dinv = dinv_r[...]
        z = jnp.dot(s_r[0] + s_r[1] + p_r[...], w2_r[...],
                    preferred_element_type=F32)
        pre = dinv * z + b_r[...]
        h = _elu(_ln(pre, g_r[...], be_r[...]))
        o_r[...] = dinv * jnp.dot(h, w3_r[...], preferred_element_type=F32)

    return pl.pallas_call(
        body,
        grid=(NPAD // BN,),
        in_specs=[_row_spec(1), _stack_spec(), _row_spec(128),
                  _full_spec((1, 256)), _full_spec((1, 256)),
                  _full_spec((1, 256)), _full_spec((128, 256)),
                  _full_spec((256, 128))],
        out_specs=_row_spec(128),
        out_shape=jax.ShapeDtypeStruct((NPAD, 128), F32),
    )(dinv_col, s2, p2, b2, g2, be2, W2, W3)


def _tc4(dinv_col, s3, p3, b3, g3, be3, Wl1, bl1, g4, be4, Wl2, bl2):
    def body(dinv_r, s_r, p_r, b_r, g_r, be_r, w1_r, b1_r, g4_r, be4_r,
             w2_r, b2_r, o_r):
        dinv = dinv_r[...]
        pre = dinv * (s_r[0] + s_r[1] + p_r[...]) + b_r[...]
        h = _elu(_ln(pre, g_r[...], be_r[...]))
        h = jnp.dot(h, w1_r[...], preferred_element_type=F32) + b1_r[...]
        h = _elu(_ln(h, g4_r[...], be4_r[...]))
        o_r[...] = jnp.dot(h, w2_r[...], preferred_element_type=F32) + b2_r[...]

    return pl.pallas_call(
        body,
        grid=(NPAD // BN,),
        in_specs=[_row_spec(1), _stack_spec(), _row_spec(128),
                  _full_spec((1, 128)), _full_spec((1, 128)),
                  _full_spec((1, 128)), _full_spec((128, 64)),
                  _full_spec((1, 64)), _full_spec((1, 64)),
                  _full_spec((1, 64)), _full_spec((64, 64)),
                  _full_spec((1, 64))],
        out_specs=_row_spec(64),
        out_shape=jax.ShapeDtypeStruct((NPAD, 64), F32),
    )(dinv_col, s3, p3, b3, g3, be3, Wl1, bl1, g4, be4, Wl2, bl2)


def kernel(x, edge_index, W1, b1, g1, be1, W2, b2, g2, be2, W3, b3, g3, be3,
           Wl1, bl1, g4, be4, Wl2, bl2):
    pad = EPAD - E
    pad_idx = N + (jnp.arange(pad, dtype=jnp.int32) % 128)
    src1d = jnp.concatenate([edge_index[0], pad_idx])
    dst1d = jnp.concatenate([edge_index[1], pad_idx])
    dst2d = dst1d.reshape(NROWS, 128)
    zeros2d = jnp.zeros((128, 128), F32)

    row = lambda v: v.reshape(1, -1)
    col = lambda v: v.reshape(NPAD, 1)

    t1 = _tc_mm(x, W1)
    deg0, deg1 = _deg_kernel(dst2d)
    dinv_col, p1 = _tc1(col(deg0), col(deg1), t1)
    s1 = _conv_kernel(p1, src1d, dst1d, zeros2d)
    p2 = _tc2(dinv_col, s1, p1, row(b1), row(g1), row(be1))
    s2 = _conv_kernel(p2, src1d, dst1d, zeros2d)
    p3 = _tc3(dinv_col, s2, p2, row(b2), row(g2), row(be2), W2, W3)
    s3 = _conv_kernel(p3, src1d, dst1d, zeros2d)
    out = _tc4(dinv_col, s3, p3, row(b3), row(g3), row(be3),
               Wl1, row(bl1), row(g4), row(be4), Wl2, row(bl2))
    return out[:N]
